# baseline (speedup 1.0000x reference)
import functools

import jax
import jax.numpy as jnp
import numpy as np
from jax import lax
from jax.experimental import pallas as pl
from jax.experimental.pallas import tpu as pltpu
from jax.experimental.pallas import tpu_sc as plsc

TOKEN_SPARSITY = 0.3

_IMIN = np.int32(np.uint32(0x80000000))
_BITS = [np.int32(np.uint32(1 << b)) for b in range(31, -1, -1)]
_CBITS = [np.int32(1 << b) for b in range(12, -1, -1)]


def _mm_kernel(x_ref, w_ref, b_ref, tv_ref, lim_ref, pos_ref, *, ni, nj, T, F):
    i, j = pl.program_id(0), pl.program_id(1)
    acc = lax.dot_general(
        x_ref[...], w_ref[...],
        dimension_numbers=(((1,), (1,)), ((), ())),
        preferred_element_type=jnp.float32,
    )
    tv = acc + b_ref[...]
    tv_ref[...] = tv
    npos = jnp.sum((tv > 0.0).astype(jnp.int32))

    @pl.when((i == 0) & (j == 0))
    def _():
        pos_ref[0] = 0

    pos_ref[0] += npos

    @pl.when((i == ni - 1) & (j == nj - 1))
    def _():
        total = pos_ref[0].astype(jnp.float32)
        lim = jnp.floor(jnp.float32(TOKEN_SPARSITY) * total / jnp.float32(T))
        lim_ref[0, 0] = jnp.clip(lim.astype(jnp.int32), 0, F)


def _matmul_limit(x, weight, bias2d):
    T, D = x.shape
    F = weight.shape[0]
    bi, bj = 4096, 512
    ni, nj = T // bi, F // bj
    return pl.pallas_call(
        functools.partial(_mm_kernel, ni=ni, nj=nj, T=T, F=F),
        grid=(ni, nj),
        in_specs=[
            pl.BlockSpec((bi, D), lambda i, j: (i, 0)),
            pl.BlockSpec((bj, D), lambda i, j: (j, 0)),
            pl.BlockSpec((1, bj), lambda i, j: (0, j)),
        ],
        out_specs=[
            pl.BlockSpec((bi, bj), lambda i, j: (i, j)),
            pl.BlockSpec(memory_space=pltpu.SMEM),
        ],
        out_shape=[
            jax.ShapeDtypeStruct((T, F), jnp.float32),
            jax.ShapeDtypeStruct((1, 1), jnp.int32),
        ],
        scratch_shapes=[pltpu.SMEM((1,), jnp.int32)],
    )(x, weight, bias2d)


def _count_kernel(lim_ref, tv_ref, cnt_ref, ps_ref, *, F):
    i = pl.program_id(0)
    k = lim_ref[0, 0]
    tv = tv_ref[...]
    R = tv.shape[0]
    bits = lax.bitcast_convert_type(tv, jnp.int32)
    key = jnp.where(bits < 0, bits ^ np.int32(0x7FFFFFFF), bits)
    key = jnp.where(tv == 0.0, 0, key)

    ones = jnp.ones((F, 128), jnp.float32)

    def row_count(mask):
        s = lax.dot_general(mask.astype(jnp.float32), ones,
                            dimension_numbers=(((1,), (0,)), ((), ())),
                            preferred_element_type=jnp.float32)
        return s[:, :1].astype(jnp.int32)

    ub = jnp.zeros((R, 1), jnp.int32)
    for m in _BITS:
        cand = ub | m
        cs = cand ^ _IMIN
        ub = jnp.where(row_count(key >= cs) >= k, cand, ub)
    thr = ub ^ _IMIN

    gt = key > thr
    eq = key == thr
    c_gt = row_count(gt)
    c_ge = c_gt + row_count(eq)

    ps_ref[...] = jnp.sum((gt | eq).astype(jnp.int32), axis=0, keepdims=True)

    @pl.when(jnp.any(c_ge != k))
    def _():
        allowed = k - c_gt
        col = lax.broadcasted_iota(jnp.int32, (R, F), 1)
        tb = jnp.zeros((R, 1), jnp.int32)
        for m in _CBITS:
            cand = tb | m
            c2 = row_count(eq & (col < cand))
            tb = jnp.where(c2 <= allowed, cand, tb)
        counted = gt | (eq & (col < tb))
        ps_ref[...] = jnp.sum(counted.astype(jnp.int32), axis=0, keepdims=True)

    @pl.when(i == 0)
    def _():
        cnt_ref[...] = jnp.zeros_like(cnt_ref)

    cnt_ref[...] += ps_ref[...]


def _neuron_counts(true_value, limit):
    T, F = true_value.shape
    bi = 256
    ni = T // bi
    return pl.pallas_call(
        functools.partial(_count_kernel, F=F),
        grid=(ni,),
        in_specs=[
            pl.BlockSpec(memory_space=pltpu.SMEM),
            pl.BlockSpec((bi, F), lambda i: (i, 0)),
        ],
        out_specs=pl.BlockSpec((1, F), lambda i: (0, 0)),
        out_shape=jax.ShapeDtypeStruct((1, F), jnp.int32),
        scratch_shapes=[pltpu.VMEM((1, F), jnp.int32)],
    )(limit, true_value)


def _rank_kernel(ccol_ref, crow_ref, rank_ref, *, F, bi):
    i = pl.program_id(0)
    R = ccol_ref.shape[0]
    iota_j = lax.broadcasted_iota(jnp.int32, (1, F), 1)
    comb_j = crow_ref[...] * F + (F - 1) - iota_j
    iota_i = lax.broadcasted_iota(jnp.int32, (R, 1), 0) + i * bi
    comb_i = ccol_ref[...] * F + (F - 1) - iota_i
    rank_ref[...] = jnp.sum((comb_j > comb_i).astype(jnp.int32), axis=1,
                            keepdims=True)


def _neuron_ranks(counts_col, counts_row):
    F = counts_col.shape[0]
    bi = 256
    ni = F // bi
    return pl.pallas_call(
        functools.partial(_rank_kernel, F=F, bi=bi),
        grid=(ni,),
        in_specs=[
            pl.BlockSpec((bi, 1), lambda i: (i, 0)),
            pl.BlockSpec((1, F), lambda i: (0, 0)),
        ],
        out_specs=pl.BlockSpec((bi, 1), lambda i: (i, 0)),
        out_shape=jax.ShapeDtypeStruct((F, 1), jnp.int32),
    )(counts_col, counts_row)


def _sc_gather(ranks, weight, bias):
    F, D = weight.shape
    R = F // 2
    info = plsc.get_sparse_core_info()
    NC, NS, L = info.num_cores, info.num_subcores, info.num_lanes
    NW = NC * NS
    rows_per_w = R // NW
    mesh = plsc.VectorSubcoreMesh(core_axis_name="c", subcore_axis_name="s")

    @functools.partial(
        pl.kernel,
        mesh=mesh,
        compiler_params=pltpu.CompilerParams(needs_layout_passes=False),
        out_type=[
            jax.ShapeDtypeStruct((R, D), jnp.float32),
            jax.ShapeDtypeStruct((R,), jnp.float32),
        ],
        scratch_types=[
            pltpu.VMEM((F,), jnp.int32),
            pltpu.VMEM((R,), jnp.int32),
            pltpu.VMEM((rows_per_w,), jnp.int32),
            pltpu.VMEM((rows_per_w, D), jnp.float32),
            pltpu.VMEM((F,), jnp.float32),
            pltpu.VMEM((rows_per_w,), jnp.float32),
            pltpu.SemaphoreType.DMA,
        ],
    )
    def k(ranks_hbm, w_hbm, b_hbm, fw_hbm, fb_hbm,
          ranks_v, inv_v, idx_v, rows_v, bias_v, fb_v, sem):
        wid = lax.axis_index("s") * NC + lax.axis_index("c")
        base = wid * rows_per_w
        pltpu.sync_copy(ranks_hbm, ranks_v)
        pltpu.sync_copy(b_hbm, bias_v)

        def body(t, carry):
            ids = lax.iota(jnp.int32, L) + t * L
            r = ranks_v[pl.ds(t * L, L)]
            plsc.store_scatter(inv_v, [r], ids, mask=r < R)
            return carry

        lax.fori_loop(0, F // L, body, 0)

        for t in range(rows_per_w // L):
            idx_v[pl.ds(t * L, L)] = inv_v[pl.ds(base + t * L, L)]

        pltpu.async_copy(w_hbm.at[idx_v], rows_v, sem).wait()
        pltpu.sync_copy(rows_v, fw_hbm.at[pl.ds(base, rows_per_w)])

        for t in range(rows_per_w // L):
            ii = idx_v[pl.ds(t * L, L)]
            fb_v[pl.ds(t * L, L)] = plsc.load_gather(bias_v, [ii])
        pltpu.sync_copy(fb_v, fb_hbm.at[pl.ds(base, rows_per_w)])

    return k(ranks, weight, bias)


def kernel(x, weight, bias):
    T, D = x.shape
    F = weight.shape[0]
    bias2d = bias.reshape(1, F)
    true_value, limit = _matmul_limit(x, weight, bias2d)
    counts = _neuron_counts(true_value, limit)
    ranks = _neuron_ranks(counts.reshape(F, 1), counts)
    filtered_W, filtered_bias = _sc_gather(ranks.reshape(F), weight, bias)
    return (true_value, filtered_W, filtered_bias)


# speedup vs baseline: 2.5858x; 2.5858x over previous
import functools

import jax
import jax.numpy as jnp
import numpy as np
from jax import lax
from jax.experimental import pallas as pl
from jax.experimental.pallas import tpu as pltpu
from jax.experimental.pallas import tpu_sc as plsc

TOKEN_SPARSITY = 0.3

_IMIN = np.int32(np.uint32(0x80000000))
_BITS = [np.int32(np.uint32(1 << b)) for b in range(31, -1, -1)]
_CBITS = [np.int32(1 << b) for b in range(12, -1, -1)]


def _mm_kernel(x_ref, w_ref, b_ref, tv_ref, lim_ref, pos_ref, *, ni, T, F):
    i = pl.program_id(0)
    acc = lax.dot_general(
        x_ref[...], w_ref[...],
        dimension_numbers=(((1,), (1,)), ((), ())),
        preferred_element_type=jnp.float32,
    )
    tv = acc + b_ref[...]
    tv_ref[...] = tv
    npos = jnp.sum((tv > 0.0).astype(jnp.int32))

    @pl.when(i == 0)
    def _():
        pos_ref[0] = 0

    pos_ref[0] += npos

    @pl.when(i == ni - 1)
    def _():
        total = pos_ref[0].astype(jnp.float32)
        lim = jnp.floor(jnp.float32(TOKEN_SPARSITY) * total / jnp.float32(T))
        lim_ref[0, 0] = jnp.clip(lim.astype(jnp.int32), 0, F)


def _matmul_limit(x, weight, bias2d):
    T, D = x.shape
    F = weight.shape[0]
    bi = 512
    ni = T // bi
    return pl.pallas_call(
        functools.partial(_mm_kernel, ni=ni, T=T, F=F),
        grid=(ni,),
        in_specs=[
            pl.BlockSpec((bi, D), lambda i: (i, 0)),
            pl.BlockSpec((F, D), lambda i: (0, 0)),
            pl.BlockSpec((1, F), lambda i: (0, 0)),
        ],
        out_specs=[
            pl.BlockSpec((bi, F), lambda i: (i, 0)),
            pl.BlockSpec(memory_space=pltpu.SMEM),
        ],
        out_shape=[
            jax.ShapeDtypeStruct((T, F), jnp.float32),
            jax.ShapeDtypeStruct((1, 1), jnp.int32),
        ],
        scratch_shapes=[pltpu.SMEM((1,), jnp.int32)],
    )(x, weight, bias2d)


def _count_kernel(lim_ref, tv_ref, cnt_ref, ps_ref, *, F):
    i = pl.program_id(0)
    k = lim_ref[0, 0]
    tv = tv_ref[...]
    R = tv.shape[0]
    bits = lax.bitcast_convert_type(tv, jnp.int32)
    key = jnp.where(bits < 0, bits ^ np.int32(0x7FFFFFFF), bits)
    key = jnp.where(tv == 0.0, 0, key)

    def row_count(mask):
        return jnp.sum(mask.astype(jnp.int32), axis=1, keepdims=True)

    ub = jnp.zeros((R, 1), jnp.int32)
    for m in _BITS:
        cand = ub | m
        cs = cand ^ _IMIN
        ub = jnp.where(row_count(key >= cs) >= k, cand, ub)
    thr = ub ^ _IMIN

    gt = key > thr
    eq = key == thr
    c_gt = row_count(gt)
    c_ge = c_gt + row_count(eq)

    ps_ref[...] = jnp.sum((gt | eq).astype(jnp.int32), axis=0, keepdims=True)

    @pl.when(jnp.any(c_ge != k))
    def _():
        allowed = k - c_gt
        col = lax.broadcasted_iota(jnp.int32, (R, F), 1)
        tb = jnp.zeros((R, 1), jnp.int32)
        for m in _CBITS:
            cand = tb | m
            c2 = row_count(eq & (col < cand))
            tb = jnp.where(c2 <= allowed, cand, tb)
        counted = gt | (eq & (col < tb))
        ps_ref[...] = jnp.sum(counted.astype(jnp.int32), axis=0, keepdims=True)

    @pl.when(i == 0)
    def _():
        cnt_ref[...] = jnp.zeros_like(cnt_ref)

    cnt_ref[...] += ps_ref[...]


def _neuron_counts(true_value, limit):
    T, F = true_value.shape
    bi = 256
    ni = T // bi
    return pl.pallas_call(
        functools.partial(_count_kernel, F=F),
        grid=(ni,),
        in_specs=[
            pl.BlockSpec(memory_space=pltpu.SMEM),
            pl.BlockSpec((bi, F), lambda i: (i, 0)),
        ],
        out_specs=pl.BlockSpec((1, F), lambda i: (0, 0)),
        out_shape=jax.ShapeDtypeStruct((1, F), jnp.int32),
        scratch_shapes=[pltpu.VMEM((1, F), jnp.int32)],
    )(limit, true_value)


def _rank_kernel(ccol_ref, crow_ref, rank_ref, *, F, bi):
    i = pl.program_id(0)
    R = ccol_ref.shape[0]
    iota_j = lax.broadcasted_iota(jnp.int32, (1, F), 1)
    comb_j = crow_ref[...] * F + (F - 1) - iota_j
    iota_i = lax.broadcasted_iota(jnp.int32, (R, 1), 0) + i * bi
    comb_i = ccol_ref[...] * F + (F - 1) - iota_i
    rank_ref[...] = jnp.sum((comb_j > comb_i).astype(jnp.int32), axis=1,
                            keepdims=True)


def _neuron_ranks(counts_col, counts_row):
    F = counts_col.shape[0]
    bi = 256
    ni = F // bi
    return pl.pallas_call(
        functools.partial(_rank_kernel, F=F, bi=bi),
        grid=(ni,),
        in_specs=[
            pl.BlockSpec((bi, 1), lambda i: (i, 0)),
            pl.BlockSpec((1, F), lambda i: (0, 0)),
        ],
        out_specs=pl.BlockSpec((bi, 1), lambda i: (i, 0)),
        out_shape=jax.ShapeDtypeStruct((F, 1), jnp.int32),
    )(counts_col, counts_row)


def _sc_gather(ranks, weight, bias):
    F, D = weight.shape
    R = F // 2
    info = plsc.get_sparse_core_info()
    NC, NS, L = info.num_cores, info.num_subcores, info.num_lanes
    NW = NC * NS
    rows_per_w = R // NW
    mesh = plsc.VectorSubcoreMesh(core_axis_name="c", subcore_axis_name="s")

    @functools.partial(
        pl.kernel,
        mesh=mesh,
        compiler_params=pltpu.CompilerParams(needs_layout_passes=False),
        out_type=[
            jax.ShapeDtypeStruct((R, D), jnp.float32),
            jax.ShapeDtypeStruct((R,), jnp.float32),
        ],
        scratch_types=[
            pltpu.VMEM((F,), jnp.int32),
            pltpu.VMEM((R,), jnp.int32),
            pltpu.VMEM((rows_per_w,), jnp.int32),
            pltpu.VMEM((rows_per_w, D), jnp.float32),
            pltpu.VMEM((F,), jnp.float32),
            pltpu.VMEM((rows_per_w,), jnp.float32),
            pltpu.SemaphoreType.DMA,
        ],
    )
    def k(ranks_hbm, w_hbm, b_hbm, fw_hbm, fb_hbm,
          ranks_v, inv_v, idx_v, rows_v, bias_v, fb_v, sem):
        wid = lax.axis_index("s") * NC + lax.axis_index("c")
        base = wid * rows_per_w
        pltpu.sync_copy(ranks_hbm, ranks_v)
        pltpu.sync_copy(b_hbm, bias_v)

        def body(t, carry):
            ids = lax.iota(jnp.int32, L) + t * L
            r = ranks_v[pl.ds(t * L, L)]
            plsc.store_scatter(inv_v, [r], ids, mask=r < R)
            return carry

        lax.fori_loop(0, F // L, body, 0)

        for t in range(rows_per_w // L):
            idx_v[pl.ds(t * L, L)] = inv_v[pl.ds(base + t * L, L)]

        pltpu.async_copy(w_hbm.at[idx_v], rows_v, sem).wait()
        pltpu.sync_copy(rows_v, fw_hbm.at[pl.ds(base, rows_per_w)])

        for t in range(rows_per_w // L):
            ii = idx_v[pl.ds(t * L, L)]
            fb_v[pl.ds(t * L, L)] = plsc.load_gather(bias_v, [ii])
        pltpu.sync_copy(fb_v, fb_hbm.at[pl.ds(base, rows_per_w)])

    return k(ranks, weight, bias)


def kernel(x, weight, bias):
    T, D = x.shape
    F = weight.shape[0]
    bias2d = bias.reshape(1, F)
    true_value, limit = _matmul_limit(x, weight, bias2d)
    counts = _neuron_counts(true_value, limit)
    ranks = _neuron_ranks(counts.reshape(F, 1), counts)
    filtered_W, filtered_bias = _sc_gather(ranks.reshape(F), weight, bias)
    return (true_value, filtered_W, filtered_bias)
